# baseline (speedup 1.0000x reference)
import functools

import jax
import jax.numpy as jnp
from jax import lax
from jax.experimental import pallas as pl
from jax.experimental.pallas import tpu as pltpu
from jax.experimental.pallas import tpu_sc as plsc

_L = 16


_WIN = 512


@functools.cache
def _build_transpose(V, D, dtype):
    info = plsc.get_sparse_core_info()
    NC, NS = info.num_cores, info.num_subcores
    NW = NC * NS
    nfull = (V // _WIN)
    tail_start = nfull * _WIN
    tail_len = V - tail_start
    hw = _WIN // 4
    mesh = plsc.VectorSubcoreMesh(core_axis_name="c", subcore_axis_name="s")

    @functools.partial(
        pl.kernel,
        mesh=mesh,
        compiler_params=pltpu.CompilerParams(needs_layout_passes=False),
        out_type=jax.ShapeDtypeStruct((V, D), dtype),
        scratch_types=[
            pltpu.VMEM((2 * D, _WIN), dtype),
            pltpu.VMEM((2 * hw, D), dtype),
            pltpu.SemaphoreType.DMA,
            pltpu.SemaphoreType.DMA,
        ],
    )
    def k(tabT_hbm, tail_hbm, out_hbm, win_v, slab_v, wsem, ssem):
        wid = lax.axis_index("s") * NC + lax.axis_index("c")
        wlo = (nfull * wid) // NW
        whi = (nfull * (wid + 1)) // NW
        lanes = lax.iota(jnp.int32, _L)

        def fetch_win(win):
            par = (win - wlo) % 2
            s = pl.multiple_of(win * _WIN, _WIN)
            pltpu.async_copy(tabT_hbm.at[:, pl.ds(s, _WIN)],
                             win_v.at[pl.ds(par * D, D)], wsem)

        @pl.when(whi > wlo)
        def _():
            fetch_win(wlo)

        @pl.when(wid == NW - 1)
        def _():
            pltpu.sync_copy(tail_hbm, out_hbm.at[pl.ds(tail_start,
                                                       tail_len)])

        def per_window(win, _):
            par = (win - wlo) % 2
            pltpu.make_async_copy(tabT_hbm.at[:, pl.ds(0, _WIN)],
                                  win_v.at[pl.ds(par * D, D)],
                                  wsem).wait()

            @pl.when(win + 1 < whi)
            def _():
                fetch_win(win + 1)

            for s in range(4):
                sp = s % 2

                @pl.when((win - wlo >= 1) | (s >= 2))
                def _(sp=sp):
                    pltpu.make_async_copy(
                        slab_v.at[pl.ds(sp * hw, hw)],
                        out_hbm.at[pl.ds(0, hw)], ssem).wait()

                def transpose(cc, _, s=s, sp=sp, par=par):
                    cols = s * hw + cc * _L + lanes
                    rows_out = sp * hw + cc * _L + lanes
                    for d in range(D):
                        vals = plsc.load_gather(
                            win_v, [jnp.full((_L,), par * D + d,
                                             jnp.int32), cols])
                        plsc.store_scatter(
                            slab_v, [rows_out,
                                     jnp.full((_L,), d, jnp.int32)],
                            vals)
                    return ()

                lax.fori_loop(0, hw // _L, transpose, (), unroll=False)
                base = pl.multiple_of(win * _WIN + s * hw, hw)
                pltpu.async_copy(slab_v.at[pl.ds(sp * hw, hw)],
                                 out_hbm.at[pl.ds(base, hw)], ssem)
            return ()

        lax.fori_loop(wlo, whi, per_window, (), unroll=False)

        @pl.when(whi > wlo)
        def _():
            def fin(u, _):
                pltpu.make_async_copy(slab_v.at[pl.ds(0, hw)],
                                      out_hbm.at[pl.ds(0, hw)],
                                      ssem).wait()
                return ()

            lax.fori_loop(0, 2, fin, (), unroll=False)

    return k


@functools.cache
def _build(B, D, dtype):
    info = plsc.get_sparse_core_info()
    NC, NS = info.num_cores, info.num_subcores
    NW = NC * NS
    b_per_w = B // NW
    mesh = plsc.VectorSubcoreMesh(core_axis_name="c", subcore_axis_name="s")

    @functools.partial(
        pl.kernel,
        mesh=mesh,
        out_type=jax.ShapeDtypeStruct((B, D), dtype),
        scratch_types=[
            pltpu.VMEM((b_per_w,), jnp.int32),
            pltpu.VMEM((b_per_w, D), dtype),
            pltpu.SemaphoreType.DMA,
        ],
    )
    def k(idx_hbm, tab_hbm, out_hbm, idx_v, rows_v, sem):
        wid = lax.axis_index("s") * NC + lax.axis_index("c")
        base = wid * b_per_w
        pltpu.sync_copy(idx_hbm.at[pl.ds(base, b_per_w)], idx_v)

        def fetch(c, _):
            vec = idx_v[pl.ds(c * _L, _L)]
            for j in range(_L):
                i = c * _L + j
                r = vec[j]
                pltpu.async_copy(tab_hbm.at[pl.ds(r, 1)],
                                 rows_v.at[pl.ds(i, 1)], sem)
            return ()

        lax.fori_loop(0, b_per_w // _L, fetch, (), unroll=False)
        pltpu.make_async_copy(tab_hbm.at[pl.ds(0, b_per_w)], rows_v,
                              sem).wait()
        pltpu.sync_copy(rows_v, out_hbm.at[pl.ds(base, b_per_w)])

    return k


def kernel(users, items, user_table, item_table):
    B = users.shape[0]
    V, D = user_table.shape
    tail_start = (V // _WIN) * _WIN
    tk = _build_transpose(V, D, user_table.dtype)
    utab = tk(user_table.T, user_table[tail_start:])
    itab = tk(item_table.T, item_table[tail_start:])
    k = _build(B, D, user_table.dtype)
    users_embs = k(users.astype(jnp.int32), utab)
    items_embs = k(items.astype(jnp.int32), itab)
    return (users_embs, items_embs)


# speedup vs baseline: 3.2708x; 3.2708x over previous
import functools

import jax
import jax.numpy as jnp
from jax import lax
from jax.experimental import pallas as pl
from jax.experimental.pallas import tpu as pltpu
from jax.experimental.pallas import tpu_sc as plsc

_L = 16


@functools.cache
def _build(B, D, dtype):
    info = plsc.get_sparse_core_info()
    NC, NS = info.num_cores, info.num_subcores
    NW = NC * NS
    b_per_w = B // NW
    mesh = plsc.VectorSubcoreMesh(core_axis_name="c", subcore_axis_name="s")

    @functools.partial(
        pl.kernel,
        mesh=mesh,
        out_type=jax.ShapeDtypeStruct((B, D), dtype),
        scratch_types=[
            pltpu.VMEM((b_per_w,), jnp.int32),
            pltpu.VMEM((b_per_w, D), dtype),
            pltpu.SemaphoreType.DMA,
        ],
    )
    def k(idx_hbm, tab_hbm, out_hbm, idx_v, rows_v, sem):
        wid = lax.axis_index("s") * NC + lax.axis_index("c")
        base = wid * b_per_w
        pltpu.sync_copy(idx_hbm.at[pl.ds(base, b_per_w)], idx_v)

        def fetch(c, _):
            vec = idx_v[pl.ds(c * _L, _L)]
            for j in range(_L):
                i = c * _L + j
                r = vec[j]
                pltpu.async_copy(tab_hbm.at[pl.ds(r, 1)],
                                 rows_v.at[pl.ds(i, 1)], sem)
            return ()

        lax.fori_loop(0, b_per_w // _L, fetch, (), unroll=False)
        pltpu.make_async_copy(tab_hbm.at[pl.ds(0, b_per_w)], rows_v,
                              sem).wait()
        pltpu.sync_copy(rows_v, out_hbm.at[pl.ds(base, b_per_w)])

    return k


def kernel(users, items, user_table, item_table):
    B = users.shape[0]
    D = user_table.shape[1]
    k = _build(B, D, user_table.dtype)
    users_embs = k(users.astype(jnp.int32), user_table)
    items_embs = k(items.astype(jnp.int32), item_table)
    return (users_embs, items_embs)
